# baseline (speedup 1.0000x reference)
import functools

import jax
import jax.numpy as jnp
from jax import lax
from jax.experimental import pallas as pl
from jax.experimental.pallas import tpu as pltpu
from jax.experimental.pallas import tpu_sc as plsc

N = 819200
B = 16384
DIM = 3
PAD = 4
NC = 2
NS = 16
NW = NC * NS
C = N // NW
K = 6400
GROUPS = K // 16
BAGS_W = B // NW
NCH = C // K
WSZ = 16384

_mesh = plsc.VectorSubcoreMesh(core_axis_name="c", subcore_axis_name="s")


@functools.partial(
    pl.kernel,
    mesh=_mesh,
    out_type=[
        jax.ShapeDtypeStruct((N * PAD,), jnp.float32),
        jax.ShapeDtypeStruct((B * PAD,), jnp.float32),
    ],
    scratch_types=[
        pltpu.VMEM((32,), jnp.float32),
        pltpu.VMEM((32,), jnp.float32),
        pltpu.VMEM((K,), jnp.int32),
        pltpu.VMEM((K,), jnp.int32),
        pltpu.VMEM((K * PAD,), jnp.float32),
        pltpu.VMEM((K * PAD,), jnp.float32),
        pltpu.VMEM((BAGS_W + 16,), jnp.int32),
        pltpu.VMEM((WSZ,), jnp.int32),
        pltpu.VMEM((BAGS_W * PAD,), jnp.float32),
        pltpu.VMEM((16,), jnp.float32),
        pltpu.SemaphoreType.DMA,
        pltpu.SemaphoreType.DMA,
        pltpu.SemaphoreType.DMA,
        pltpu.SemaphoreType.DMA,
    ],
    compiler_params=pltpu.CompilerParams(needs_layout_passes=False),
)
def _sc_kernel(data_hbm, offs_hbm, w1_hbm, w2_hbm, x_hbm, x2_hbm,
               w1_v, w2_v, idx0_v, idx1_v, xout0_v, xout1_v, offs_v, win_v,
               bag_v, hist_v, isem0, isem1, osem0, osem1):
    wid = lax.axis_index("s") * NC + lax.axis_index("c")
    iota = lax.iota(jnp.int32, 16)

    pltpu.sync_copy(w1_hbm, w1_v)
    pltpu.sync_copy(w2_hbm, w2_v)

    base_elem = wid * C
    idx_b = (idx0_v, idx1_v)
    xout_b = (xout0_v, xout1_v)
    isems = (isem0, isem1)
    osems = (osem0, osem1)

    def in_start(ci):
        cstart = pl.multiple_of(base_elem + ci * K, 128)
        return pltpu.async_copy(data_hbm.at[pl.ds(cstart, K)],
                                idx_b[ci % 2], isems[ci % 2])

    in_h = [None] * NCH
    out_h = [None] * NCH
    in_h[0] = in_start(0)
    for ci in range(NCH):
        cur = ci % 2
        if ci + 1 < NCH:
            in_h[ci + 1] = in_start(ci + 1)
        in_h[ci].wait()
        if ci >= 2:
            out_h[ci - 2].wait()
        ix = idx_b[cur]
        xo = xout_b[cur]

        @plsc.parallel_loop(0, GROUPS, unroll=4)
        def grp(g):
            d = ix[pl.ds(g * 16, 16)]
            d3 = d * 3
            gbase = (g // 8) * (PAD * 128) + (g % 8) * 16
            for comp in range(DIM):
                xo[pl.ds(gbase + comp * 128, 16)] = plsc.load_gather(
                    w1_v, [d3 + comp])

        cstart = pl.multiple_of(base_elem + ci * K, 128)
        out_h[ci] = pltpu.async_copy(
            xo, x_hbm.at[pl.ds(cstart * PAD, K * PAD)], osems[cur])
    out_h[NCH - 2].wait()
    out_h[NCH - 1].wait()

    b0 = pl.multiple_of(wid * BAGS_W, 128)
    pltpu.sync_copy(offs_hbm.at[pl.ds(b0, BAGS_W)], offs_v.at[pl.ds(0, BAGS_W)])
    probe = pl.multiple_of(jnp.minimum(b0 + BAGS_W, B - 8), 8)
    pltpu.sync_copy(offs_hbm.at[pl.ds(probe, 8)], offs_v.at[pl.ds(BAGS_W, 8)])
    zero16 = jnp.zeros((16,), jnp.float32)
    ones16 = jnp.ones((16,), jnp.float32)
    last_worker = wid == NW - 1
    w2c = [plsc.load_gather(w2_v, [jnp.minimum(iota * 3 + comp, 31)])
           for comp in range(DIM)]
    hist_v[...] = zero16

    @plsc.parallel_loop(0, BAGS_W * PAD, step=16)
    def zero_body(g):
        bag_v[pl.ds(g, 16)] = zero16

    def bag_body(b, wbase):
        se = plsc.load_gather(offs_v, [b + iota])
        s = se[0]
        e = jnp.where(last_worker & (b == BAGS_W - 1), N, se[1])

        def cond(carry):
            k, _ = carry
            return k < e

        def body(carry):
            k, wb = carry
            need = (k + 16) > (wb + WSZ)
            wb2 = jnp.where(need, jnp.minimum((k // 8) * 8, N - WSZ), wb)

            @pl.when(need)
            def _():
                pltpu.sync_copy(
                    data_hbm.at[pl.ds(pl.multiple_of(wb2, 8), WSZ)], win_v)

            li = jnp.minimum(k + iota, e - 1) - wb2
            d = plsc.load_gather(win_v, [li])
            plsc.addupdate_scatter(hist_v, [d], ones16,
                                   mask=(k + iota) < e)
            return (k + 16, wb2)

        _, wb_fin = lax.while_loop(cond, body, (s, wbase))
        h = hist_v[...]
        hist_v[...] = zero16
        bbase = (b // 128) * (PAD * 128) + (b % 128)
        for comp in range(DIM):
            plsc.addupdate_scatter(
                bag_v, [jnp.broadcast_to(bbase + comp * 128, (16,))],
                h * w2c[comp])
        return wb_fin

    lax.fori_loop(0, BAGS_W, bag_body, jnp.int32(-2 ** 30))

    @plsc.parallel_loop(0, BAGS_W // 16)
    def div_body(j):
        o0 = plsc.load_gather(offs_v, [j * 16 + iota])
        o1 = plsc.load_gather(offs_v, [j * 16 + 1 + iota])
        o1 = jnp.where(last_worker & (j * 16 + iota == BAGS_W - 1), N, o1)
        inv = 1.0 / jnp.maximum((o1 - o0).astype(jnp.float32), 1.0)
        dbase = (j // 8) * (PAD * 128) + (j % 8) * 16
        for comp in range(DIM):
            sl = pl.ds(dbase + comp * 128, 16)
            bag_v[sl] = bag_v[sl] * inv

    pltpu.sync_copy(bag_v, x2_hbm.at[pl.ds(b0 * PAD, BAGS_W * PAD)])


def _untile(flat4, n):
    return (flat4.reshape(n // 128, PAD, 128)[:, :DIM, :]
            .transpose(0, 2, 1).reshape(n, DIM))


def kernel(data, offsets, W1, W2):
    w1f = jnp.concatenate([W1.reshape(-1), jnp.zeros((2,), jnp.float32)])
    w2f = jnp.concatenate([W2.reshape(-1), jnp.zeros((2,), jnp.float32)])
    xf4, x2f4 = _sc_kernel(data, offsets, w1f, w2f)
    return _untile(xf4, N), _untile(x2f4, B)


# speedup vs baseline: 1.1785x; 1.1785x over previous
import functools

import jax
import jax.numpy as jnp
from jax import lax
from jax.experimental import pallas as pl
from jax.experimental.pallas import tpu as pltpu
from jax.experimental.pallas import tpu_sc as plsc

N = 819200
B = 16384
DIM = 3
PAD = 4
NC = 2
NS = 16
NW = NC * NS
C = N // NW
K = 6400
GROUPS = K // 16
BAGS_W = B // NW
NCH = C // K
WSZ = 16384

_mesh = plsc.VectorSubcoreMesh(core_axis_name="c", subcore_axis_name="s")


@functools.partial(
    pl.kernel,
    mesh=_mesh,
    out_type=[
        jax.ShapeDtypeStruct((N * PAD,), jnp.float32),
        jax.ShapeDtypeStruct((B * PAD,), jnp.float32),
    ],
    scratch_types=[
        pltpu.VMEM((32,), jnp.float32),
        pltpu.VMEM((32,), jnp.float32),
        pltpu.VMEM((K,), jnp.int32),
        pltpu.VMEM((K,), jnp.int32),
        pltpu.VMEM((K * PAD,), jnp.float32),
        pltpu.VMEM((K * PAD,), jnp.float32),
        pltpu.VMEM((BAGS_W + 16,), jnp.int32),
        pltpu.VMEM((WSZ,), jnp.int32),
        pltpu.VMEM((BAGS_W * PAD,), jnp.float32),
        pltpu.SemaphoreType.DMA,
        pltpu.SemaphoreType.DMA,
        pltpu.SemaphoreType.DMA,
        pltpu.SemaphoreType.DMA,
    ],
    compiler_params=pltpu.CompilerParams(needs_layout_passes=False),
)
def _sc_kernel(data_hbm, offs_hbm, w1_hbm, w2_hbm, x_hbm, x2_hbm,
               w1_v, w2_v, idx0_v, idx1_v, xout0_v, xout1_v, offs_v, win_v,
               bag_v, isem0, isem1, osem0, osem1):
    wid = lax.axis_index("s") * NC + lax.axis_index("c")
    iota = lax.iota(jnp.int32, 16)

    pltpu.sync_copy(w1_hbm, w1_v)
    pltpu.sync_copy(w2_hbm, w2_v)

    base_elem = wid * C
    idx_b = (idx0_v, idx1_v)
    xout_b = (xout0_v, xout1_v)
    isems = (isem0, isem1)
    osems = (osem0, osem1)

    def in_start(ci):
        cstart = pl.multiple_of(base_elem + ci * K, 128)
        return pltpu.async_copy(data_hbm.at[pl.ds(cstart, K)],
                                idx_b[ci % 2], isems[ci % 2])

    in_h = [None] * NCH
    out_h = [None] * NCH
    in_h[0] = in_start(0)
    for ci in range(NCH):
        cur = ci % 2
        if ci + 1 < NCH:
            in_h[ci + 1] = in_start(ci + 1)
        in_h[ci].wait()
        if ci >= 2:
            out_h[ci - 2].wait()
        ix = idx_b[cur]
        xo = xout_b[cur]

        @plsc.parallel_loop(0, GROUPS, unroll=4)
        def grp(g):
            d = ix[pl.ds(g * 16, 16)]
            d3 = d * 3
            gbase = (g // 8) * (PAD * 128) + (g % 8) * 16
            for comp in range(DIM):
                xo[pl.ds(gbase + comp * 128, 16)] = plsc.load_gather(
                    w1_v, [d3 + comp])

        cstart = pl.multiple_of(base_elem + ci * K, 128)
        out_h[ci] = pltpu.async_copy(
            xo, x_hbm.at[pl.ds(cstart * PAD, K * PAD)], osems[cur])
    out_h[NCH - 2].wait()
    out_h[NCH - 1].wait()

    b0 = pl.multiple_of(wid * BAGS_W, 128)
    pltpu.sync_copy(offs_hbm.at[pl.ds(b0, BAGS_W)], offs_v.at[pl.ds(0, BAGS_W)])
    probe = pl.multiple_of(jnp.minimum(b0 + BAGS_W, B - 8), 8)
    pltpu.sync_copy(offs_hbm.at[pl.ds(probe, 8)], offs_v.at[pl.ds(BAGS_W, 8)])
    zero16 = jnp.zeros((16,), jnp.float32)
    last_worker = wid == NW - 1

    def bag_body(b, wbase):
        se = plsc.load_gather(offs_v, [b + iota])
        s = se[0]
        e = jnp.where(last_worker & (b == BAGS_W - 1), N, se[1])

        def cond(carry):
            k, _, _, _, _ = carry
            return k < e

        def body(carry):
            k, wb, a0, a1, a2 = carry
            need = (k + 16) > (wb + WSZ)
            wb2 = jnp.where(need, jnp.minimum((k // 8) * 8, N - WSZ), wb)

            @pl.when(need)
            def _():
                pltpu.sync_copy(
                    data_hbm.at[pl.ds(pl.multiple_of(wb2, 8), WSZ)], win_v)

            li = jnp.minimum(k + iota, e - 1) - wb2
            d = plsc.load_gather(win_v, [li])
            d3 = d * 3
            mask = (k + iota) < e
            a0 = a0 + jnp.where(mask, plsc.load_gather(w2_v, [d3]), 0.0)
            a1 = a1 + jnp.where(mask, plsc.load_gather(w2_v, [d3 + 1]), 0.0)
            a2 = a2 + jnp.where(mask, plsc.load_gather(w2_v, [d3 + 2]), 0.0)
            return (k + 16, wb2, a0, a1, a2)

        _, wb_fin, a0, a1, a2 = lax.while_loop(
            cond, body, (s, wbase, zero16, zero16, zero16))
        cnt_v = jnp.broadcast_to(e - s, (16,)).astype(jnp.float32)
        inv_v = 1.0 / jnp.maximum(cnt_v, 1.0)
        sums = jnp.where(iota == 0, jnp.sum(a0),
                         jnp.where(iota == 1, jnp.sum(a1), jnp.sum(a2)))
        bbase = (b // 128) * (PAD * 128) + (b % 128)
        plsc.store_scatter(bag_v, [bbase + 128 * iota], sums * inv_v,
                           mask=iota < DIM)
        return wb_fin

    lax.fori_loop(0, BAGS_W, bag_body, jnp.int32(-2 ** 30))
    pltpu.sync_copy(bag_v, x2_hbm.at[pl.ds(b0 * PAD, BAGS_W * PAD)])


def _untile(flat4, n):
    return (flat4.reshape(n // 128, PAD, 128)[:, :DIM, :]
            .transpose(0, 2, 1).reshape(n, DIM))


def kernel(data, offsets, W1, W2):
    w1f = jnp.concatenate([W1.reshape(-1), jnp.zeros((2,), jnp.float32)])
    w2f = jnp.concatenate([W2.reshape(-1), jnp.zeros((2,), jnp.float32)])
    xf4, x2f4 = _sc_kernel(data, offsets, w1f, w2f)
    return _untile(xf4, N), _untile(x2f4, B)
